# baseline (speedup 1.0000x reference)
import functools

import jax
import jax.numpy as jnp
from jax import lax
from jax.experimental import pallas as pl
from jax.experimental.pallas import tpu as pltpu
from jax.experimental.pallas import tpu_sc as plsc

_B = 4096
_T = 32
_D = 64
_P = 7
_BT = _B * _T
_N_ENT_ROWS = _P * 4 * _BT
_N_REL_ROWS = _P * 2 * _BT
_NW = 32
_K = 128
_ENT_PW = _N_ENT_ROWS // _NW
_REL_PW = _N_REL_ROWS // _NW
_ITEMS_PW = _B // _NW

_BB = 128
_NB = _B // _BB
_BBT = _BB * _T


@functools.cache
def _make_sc_gather():
    @functools.partial(
        pl.kernel,
        mesh=plsc.VectorSubcoreMesh(core_axis_name="c", subcore_axis_name="s"),
        out_type=[
            jax.ShapeDtypeStruct((_N_ENT_ROWS, _D), jnp.float32),
            jax.ShapeDtypeStruct((_B, _D), jnp.float32),
            jax.ShapeDtypeStruct((_N_REL_ROWS, _D), jnp.float32),
        ],
        scratch_types=[
            pltpu.VMEM((_K,), jnp.int32),
            pltpu.VMEM((_K, _D), jnp.float32),
            pltpu.SemaphoreType.DMA,
        ],
        compiler_params=pltpu.CompilerParams(use_tc_tiling_on_sc=False),
    )
    def _sc_gather(ent_idx, items_idx, rel_idx, ent_tab, rel_tab,
                   out_ent, out_items, out_rel, idx_v, rows_v, sem):
        wid = lax.axis_index("s") * 2 + lax.axis_index("c")

        def run(idx_hbm, tab, out_hbm, base, nchunks):
            def body(j, carry):
                o = base + j * _K
                pltpu.sync_copy(idx_hbm.at[pl.ds(o, _K)], idx_v)
                pltpu.async_copy(tab.at[idx_v], rows_v, sem).wait()
                pltpu.sync_copy(rows_v, out_hbm.at[pl.ds(o, _K)])
                return carry
            lax.fori_loop(0, nchunks, body, 0)

        run(ent_idx, ent_tab, out_ent, wid * _ENT_PW, _ENT_PW // _K)
        run(items_idx, ent_tab, out_items, wid * _ITEMS_PW, 1)
        run(rel_idx, rel_tab, out_rel, wid * _REL_PW, _REL_PW // _K)

    return _sc_gather


def _tc_body(ent_ref, rel_ref, items_ref, w1_ref, w2_ref, out_ref, acc_u, acc_v):
    p = pl.program_id(1)
    e00 = ent_ref[0, 0]
    e01 = ent_ref[0, 1]
    t0 = ent_ref[0, 2]
    t1 = ent_ref[0, 3]
    r10 = rel_ref[0, 0]
    r11 = rel_ref[0, 1]
    w1h = w1_ref[:_D, :]
    w1p = w1_ref[_D:, :]
    w2 = w2_ref[0, :]

    def layer(h, pp, t):
        s1 = jax.nn.sigmoid(
            jnp.dot(h, w1h, preferred_element_type=jnp.float32)
            + jnp.dot(pp, w1p, preferred_element_type=jnp.float32))
        a = jax.nn.sigmoid(
            jnp.sum(s1.reshape(_BB, _T, _D) * w2[None, None, :], axis=-1))
        ea = jnp.exp(a)
        att = ea / jnp.sum(ea, axis=-1, keepdims=True)
        return jnp.sum(t.reshape(_BB, _T, _D) * att[:, :, None], axis=1)

    out0 = layer(e00, r10, t0)
    out1 = layer(e00 + e01, r10 * r11, t1)
    mean0 = jnp.sum(e00.reshape(_BB, _T, _D), axis=1) * (1.0 / _T)
    base = mean0 + out0 + out1

    @pl.when(p == 0)
    def _():
        acc_u[...] = jnp.zeros_like(acc_u)
        acc_v[...] = jnp.zeros_like(acc_v)

    u_w = jnp.where(p < 4, 1.0, 0.0)
    v_w = jnp.where(p < 4, 0.0, jnp.where(p == 4, 2.0, 1.0))
    item_w = jnp.where(p == 4, 2.0, 0.0)
    acc_u[...] += u_w * base
    acc_v[...] += v_w * base + item_w * items_ref[...]

    @pl.when(p == _P - 1)
    def _():
        out_ref[...] = jax.nn.sigmoid(
            jnp.sum(acc_u[...] * acc_v[...], axis=-1))


def _tc_compute(ent4, rel2, g_items, W1, W2row):
    return pl.pallas_call(
        _tc_body,
        grid=(_NB, _P),
        in_specs=[
            pl.BlockSpec((1, 4, _BBT, _D), lambda i, p: (p, 0, i, 0)),
            pl.BlockSpec((1, 2, _BBT, _D), lambda i, p: (p, 0, i, 0)),
            pl.BlockSpec((_BB, _D), lambda i, p: (i, 0)),
            pl.BlockSpec((2 * _D, _D), lambda i, p: (0, 0)),
            pl.BlockSpec((1, _D), lambda i, p: (0, 0)),
        ],
        out_specs=pl.BlockSpec((_BB,), lambda i, p: (i,)),
        out_shape=jax.ShapeDtypeStruct((_B,), jnp.float32),
        scratch_shapes=[
            pltpu.VMEM((_BB, _D), jnp.float32),
            pltpu.VMEM((_BB, _D), jnp.float32),
        ],
    )(ent4, rel2, g_items, W1, W2row)


def kernel(items, kg_init_triple_set, ddi_potential_triple_set,
           kg_potential_triple_set, ddi_origin_triple_set,
           kg_init_triple_set1, ddi_potential_triple_set1,
           kg_potential_triple_set1, ddi_origin_triple_set1,
           embeddings_0, embeddings_1, entity_emb, relation_emb,
           W_att1, W_att2):
    procs = [kg_init_triple_set, kg_potential_triple_set,
             kg_init_triple_set1, kg_potential_triple_set1,
             ddi_potential_triple_set, ddi_potential_triple_set1,
             ddi_origin_triple_set1]
    ent_idx = jnp.stack(
        [jnp.stack([ts[0, 0], ts[0, 1], ts[2, 0], ts[2, 1]]) for ts in procs]
    ).reshape(-1)
    rel_idx = jnp.stack(
        [jnp.stack([ts[1, 0], ts[1, 1]]) for ts in procs]
    ).reshape(-1)
    g_ent, g_items, g_rel = _make_sc_gather()(
        ent_idx, items, rel_idx, entity_emb, relation_emb)
    ent4 = g_ent.reshape(_P, 4, _BT, _D)
    rel2 = g_rel.reshape(_P, 2, _BT, _D)
    return _tc_compute(ent4, rel2, g_items, W_att1, W_att2.reshape(1, _D))


# speedup vs baseline: 1.3416x; 1.3416x over previous
import functools

import jax
import jax.numpy as jnp
from jax import lax
from jax.experimental import pallas as pl
from jax.experimental.pallas import tpu as pltpu
from jax.experimental.pallas import tpu_sc as plsc

_B = 4096
_T = 32
_D = 64
_P = 7
_BT = _B * _T
_NSETS = 6 * _P
_N_ROWS = _NSETS * _BT
_NW = 32
_SPW = _B
_K = 512
_NCH = _SPW // _K
_N_ENTITY = 100000

_BB = 128
_NB = _B // _BB
_BBT = _BB * _T


@functools.cache
def _make_sc_gather():
    @functools.partial(
        pl.kernel,
        mesh=plsc.VectorSubcoreMesh(core_axis_name="c", subcore_axis_name="s"),
        out_type=[
            jax.ShapeDtypeStruct((_N_ROWS, _D), jnp.float32),
            jax.ShapeDtypeStruct((_B, _D), jnp.float32),
        ],
        scratch_types=[
            pltpu.VMEM((_SPW,), jnp.int32),
            pltpu.VMEM((_K, _D), jnp.float32),
            pltpu.VMEM((_K, _D), jnp.float32),
            pltpu.SemaphoreType.DMA,
            pltpu.SemaphoreType.DMA,
        ],
        compiler_params=pltpu.CompilerParams(use_tc_tiling_on_sc=False),
    )
    def _sc_gather(idx_all, tab, out_main, out_items,
                   idx_v, rows0, rows1, sem0, sem1):
        wid = lax.axis_index("s") * 2 + lax.axis_index("c")
        rows = (rows0, rows1)
        sems = (sem0, sem1)

        def per_set(s, carry):
            base = s * _BT + wid * _SPW
            pltpu.sync_copy(idx_all.at[pl.ds(base, _SPW)], idx_v)
            cps = [None, None]
            cps[0] = pltpu.async_copy(
                tab.at[idx_v.at[pl.ds(0, _K)]], rows[0], sems[0])
            for c in range(_NCH):
                b = c % 2
                if c + 1 < _NCH:
                    b2 = (c + 1) % 2
                    cps[b2] = pltpu.async_copy(
                        tab.at[idx_v.at[pl.ds((c + 1) * _K, _K)]],
                        rows[b2], sems[b2])
                cps[b].wait()
                pltpu.sync_copy(rows[b], out_main.at[pl.ds(base + c * _K, _K)])
            return carry

        lax.fori_loop(0, _NSETS, per_set, 0)
        ib = _N_ROWS + wid * (_B // _NW)
        pltpu.sync_copy(idx_all.at[pl.ds(ib, _B // _NW)],
                        idx_v.at[pl.ds(0, _B // _NW)])
        cp = pltpu.async_copy(
            tab.at[idx_v.at[pl.ds(0, _B // _NW)]],
            rows0.at[pl.ds(0, _B // _NW)], sem0)
        cp.wait()
        pltpu.sync_copy(rows0.at[pl.ds(0, _B // _NW)],
                        out_items.at[pl.ds(wid * (_B // _NW), _B // _NW)])

    return _sc_gather


def _tc_body(g_ref, items_ref, w1_ref, w2_ref, out_ref, acc_u, acc_v):
    p = pl.program_id(1)
    e00 = g_ref[0, 0]
    e01 = g_ref[0, 1]
    r10 = g_ref[0, 2]
    r11 = g_ref[0, 3]
    t0 = g_ref[0, 4]
    t1 = g_ref[0, 5]
    w1h = w1_ref[:_D, :]
    w1p = w1_ref[_D:, :]
    w2 = w2_ref[0, :]

    def layer(h, pp, t):
        s1 = jax.nn.sigmoid(
            jnp.dot(h, w1h, preferred_element_type=jnp.float32)
            + jnp.dot(pp, w1p, preferred_element_type=jnp.float32))
        a = jax.nn.sigmoid(
            jnp.sum(s1.reshape(_BB, _T, _D) * w2[None, None, :], axis=-1))
        ea = jnp.exp(a)
        att = ea / jnp.sum(ea, axis=-1, keepdims=True)
        return jnp.sum(t.reshape(_BB, _T, _D) * att[:, :, None], axis=1)

    out0 = layer(e00, r10, t0)
    out1 = layer(e00 + e01, r10 * r11, t1)
    mean0 = jnp.sum(e00.reshape(_BB, _T, _D), axis=1) * (1.0 / _T)
    base = mean0 + out0 + out1

    @pl.when(p == 0)
    def _():
        acc_u[...] = jnp.zeros_like(acc_u)
        acc_v[...] = jnp.zeros_like(acc_v)

    u_w = jnp.where(p < 4, 1.0, 0.0)
    v_w = jnp.where(p < 4, 0.0, jnp.where(p == 4, 2.0, 1.0))
    item_w = jnp.where(p == 4, 2.0, 0.0)
    acc_u[...] += u_w * base
    acc_v[...] += v_w * base + item_w * items_ref[...]

    @pl.when(p == _P - 1)
    def _():
        out_ref[...] = jax.nn.sigmoid(
            jnp.sum(acc_u[...] * acc_v[...], axis=-1))


def _tc_compute(g6, g_items, W1, W2row):
    return pl.pallas_call(
        _tc_body,
        grid=(_NB, _P),
        in_specs=[
            pl.BlockSpec((1, 6, _BBT, _D), lambda i, p: (p, 0, i, 0)),
            pl.BlockSpec((_BB, _D), lambda i, p: (i, 0)),
            pl.BlockSpec((2 * _D, _D), lambda i, p: (0, 0)),
            pl.BlockSpec((1, _D), lambda i, p: (0, 0)),
        ],
        out_specs=pl.BlockSpec((_BB,), lambda i, p: (i,)),
        out_shape=jax.ShapeDtypeStruct((_B,), jnp.float32),
        scratch_shapes=[
            pltpu.VMEM((_BB, _D), jnp.float32),
            pltpu.VMEM((_BB, _D), jnp.float32),
        ],
    )(g6, g_items, W1, W2row)


def kernel(items, kg_init_triple_set, ddi_potential_triple_set,
           kg_potential_triple_set, ddi_origin_triple_set,
           kg_init_triple_set1, ddi_potential_triple_set1,
           kg_potential_triple_set1, ddi_origin_triple_set1,
           embeddings_0, embeddings_1, entity_emb, relation_emb,
           W_att1, W_att2):
    procs = [kg_init_triple_set, kg_potential_triple_set,
             kg_init_triple_set1, kg_potential_triple_set1,
             ddi_potential_triple_set, ddi_potential_triple_set1,
             ddi_origin_triple_set1]
    off = jnp.array([0, _N_ENTITY, 0], jnp.int32).reshape(3, 1, 1, 1)
    idx_all = jnp.concatenate(
        [(ts + off).reshape(-1) for ts in procs] + [items])
    tab = jnp.concatenate([entity_emb, relation_emb])
    g_main, g_items = _make_sc_gather()(idx_all, tab)
    g6 = g_main.reshape(_P, 6, _BT, _D)
    return _tc_compute(g6, g_items, W_att1, W_att2.reshape(1, _D))
